# baseline (speedup 1.0000x reference)
import dataclasses
import functools

import jax
import jax.numpy as jnp
from jax import lax
from jax.experimental import pallas as pl
from jax.experimental.pallas import tpu as pltpu
from jax.experimental.pallas import tpu_sc as plsc

DROP_FRAME_PROB = 0.125
NUM_SC_CORES = 2
NUM_SC_SUBCORES = 16
LANES = 16
NBUF = 4
SUBLANES = 8


def _src_indices(T):
    rkey = jax.random.key(42)
    kdrop, kdir = jax.random.split(rkey)
    u_drop = jax.random.uniform(kdrop, (T,))
    u_dir = jax.random.uniform(kdir, (T,))
    drop = u_drop < DROP_FRAME_PROB
    diff = jnp.where(u_dir < 0.5, -1, 1)
    idx = jnp.arange(T)
    return jnp.where(drop, (idx + diff) % T, idx).astype(jnp.int32)


def _sc_scatter(aux, frames3):
    T, nsub, sub = frames3.shape
    n_workers = NUM_SC_CORES * NUM_SC_SUBCORES
    aux_len = aux.shape[0]
    max_items = (T + n_workers - 1) // n_workers

    mesh = plsc.VectorSubcoreMesh(core_axis_name="c", subcore_axis_name="s")

    cp = pltpu.CompilerParams()
    if "needs_layout_passes" in pltpu.CompilerParams.__dataclass_fields__:
        cp = dataclasses.replace(cp, needs_layout_passes=False)

    @functools.partial(
        pl.kernel,
        compiler_params=cp,
        out_type=jax.ShapeDtypeStruct((T, nsub, sub), frames3.dtype),
        mesh=mesh,
        scratch_types=[
            pltpu.VMEM((aux_len,), jnp.int32),
            pltpu.VMEM((NBUF * sub,), frames3.dtype),
            pltpu.SemaphoreType.DMA,
            pltpu.SemaphoreType.DMA((NBUF,)),
            pltpu.SemaphoreType.DMA((NBUF,)),
        ],
    )
    def scatter(aux_hbm, frames_hbm, out_hbm, aux_v, buf, sem_i, sem_in,
                sem_out):
        wid = lax.axis_index("s") * NUM_SC_CORES + lax.axis_index("c")
        pltpu.async_copy(aux_hbm, aux_v, sem_i).wait()

        iota = lax.iota(jnp.int32, LANES)
        neg = jnp.int32(-(2**31) + 1)
        vecs = [aux_v[pl.ds(j * LANES, LANES)] for j in range(aux_len // LANES)]

        def extract(pos):
            s = neg
            for j, vec in enumerate(vecs):
                m = (iota + (j * LANES)) == pos
                s = jnp.maximum(s, jnp.max(jnp.where(m, vec, neg)))
            return s

        n_drop = extract(jnp.int32(2 * T))
        for t in range(max_items):
            k = wid + t * n_workers

            @pl.when(k < n_drop)
            def _():
                d = extract(T + k)
                s = extract(d)
                in_h = [None] * nsub
                out_h = [None] * nsub

                def start_in(c):
                    in_h[c] = pltpu.async_copy(
                        frames_hbm.at[s, c],
                        buf.at[pl.ds((c % NBUF) * sub, sub)],
                        sem_in.at[c % NBUF],
                    )

                for c in range(min(NBUF - 1, nsub)):
                    start_in(c)
                for c in range(nsub):
                    in_h[c].wait()
                    out_h[c] = pltpu.async_copy(
                        buf.at[pl.ds((c % NBUF) * sub, sub)],
                        out_hbm.at[d, c],
                        sem_out.at[c % NBUF],
                    )
                    j = c + NBUF - 1
                    if j < nsub:
                        if j >= NBUF:
                            out_h[j - NBUF].wait()
                        start_in(j)
                for c in range(max(0, nsub - NBUF), nsub):
                    out_h[c].wait()

    return scatter(aux, frames3)


def kernel(frames, mask):
    T = frames.shape[0]
    row = 1
    for d in frames.shape[1:]:
        row *= d
    sub = row // SUBLANES
    frames3 = frames.reshape(T, SUBLANES, sub)
    src = _src_indices(T)

    idx = jnp.arange(T, dtype=jnp.int32)
    keep = src == idx
    n_keep = jnp.sum(keep.astype(jnp.int32))
    n_drop = jnp.int32(T) - n_keep
    big = jnp.int32(T)
    order = jnp.argsort(jnp.where(keep, idx, big + idx)).astype(jnp.int32)
    drop_order = jnp.roll(order, n_drop - big)
    first_keep = order[0]
    m = jnp.where(idx < n_keep, order, first_keep).astype(jnp.int32)

    aux_len = ((2 * T + 1 + LANES - 1) // LANES) * LANES
    aux = jnp.zeros((aux_len,), jnp.int32)
    aux = aux.at[:T].set(src).at[T:2 * T].set(drop_order)
    aux = aux.at[2 * T].set(n_drop)

    partial3 = _sc_scatter(aux, frames3)

    def tc_copy(m_ref, in_ref, partial_ref, o_ref, sem):
        del partial_ref
        handles = []
        for k in range(T):
            r = m_ref[k]
            handles.append(
                pltpu.make_async_copy(in_ref.at[r], o_ref.at[r], sem))
        for h in handles:
            h.start()
        for h in handles:
            h.wait()

    grid_spec = pltpu.PrefetchScalarGridSpec(
        num_scalar_prefetch=1,
        grid=(1,),
        in_specs=[
            pl.BlockSpec(memory_space=pl.ANY),
            pl.BlockSpec(memory_space=pl.ANY),
        ],
        out_specs=pl.BlockSpec(memory_space=pl.ANY),
        scratch_shapes=[pltpu.SemaphoreType.DMA],
    )
    out = pl.pallas_call(
        tc_copy,
        grid_spec=grid_spec,
        out_shape=jax.ShapeDtypeStruct((T, SUBLANES, sub), frames3.dtype),
        input_output_aliases={2: 0},
    )(m, frames3, partial3)

    return (out.reshape(frames.shape), mask)


# speedup vs baseline: 10.8234x; 10.8234x over previous
import dataclasses
import functools

import jax
import jax.numpy as jnp
from jax import lax
from jax.experimental import pallas as pl
from jax.experimental.pallas import tpu as pltpu
from jax.experimental.pallas import tpu_sc as plsc

DROP_FRAME_PROB = 0.125
NUM_SC_CORES = 2
NUM_SC_SUBCORES = 16
LANES = 16
NBUF = 4
SUBLANES = 8


def _src_indices(T):
    rkey = jax.random.key(42)
    kdrop, kdir = jax.random.split(rkey)
    u_drop = jax.random.uniform(kdrop, (T,))
    u_dir = jax.random.uniform(kdir, (T,))
    drop = u_drop < DROP_FRAME_PROB
    diff = jnp.where(u_dir < 0.5, -1, 1)
    idx = jnp.arange(T)
    return jnp.where(drop, (idx + diff) % T, idx).astype(jnp.int32)


def _sc_scatter(aux, frames3):
    T, nsub, sub = frames3.shape
    n_workers = NUM_SC_CORES * NUM_SC_SUBCORES
    aux_len = aux.shape[0]
    max_items = (T + n_workers - 1) // n_workers

    mesh = plsc.VectorSubcoreMesh(core_axis_name="c", subcore_axis_name="s")

    cp = pltpu.CompilerParams()
    if "needs_layout_passes" in pltpu.CompilerParams.__dataclass_fields__:
        cp = dataclasses.replace(cp, needs_layout_passes=False)

    @functools.partial(
        pl.kernel,
        compiler_params=cp,
        out_type=jax.ShapeDtypeStruct((T, nsub, sub), frames3.dtype),
        mesh=mesh,
        scratch_types=[
            pltpu.VMEM((aux_len,), jnp.int32),
            pltpu.VMEM((NBUF * sub,), frames3.dtype),
            pltpu.SemaphoreType.DMA,
            pltpu.SemaphoreType.DMA((NBUF,)),
            pltpu.SemaphoreType.DMA((NBUF,)),
        ],
    )
    def scatter(aux_hbm, frames_hbm, out_hbm, aux_v, buf, sem_i, sem_in,
                sem_out):
        wid = lax.axis_index("s") * NUM_SC_CORES + lax.axis_index("c")
        pltpu.async_copy(aux_hbm, aux_v, sem_i).wait()

        iota = lax.iota(jnp.int32, LANES)
        neg = jnp.int32(-(2**31) + 1)
        vecs = [aux_v[pl.ds(j * LANES, LANES)] for j in range(aux_len // LANES)]

        def extract(pos):
            s = neg
            for j, vec in enumerate(vecs):
                m = (iota + (j * LANES)) == pos
                s = jnp.maximum(s, jnp.max(jnp.where(m, vec, neg)))
            return s

        n_drop = extract(jnp.int32(2 * T))
        for t in range(max_items):
            k = wid + t * n_workers

            @pl.when(k < n_drop)
            def _():
                d = extract(T + k)
                s = extract(d)
                in_h = [None] * nsub
                out_h = [None] * nsub

                def start_in(c):
                    in_h[c] = pltpu.async_copy(
                        frames_hbm.at[s, c],
                        buf.at[pl.ds((c % NBUF) * sub, sub)],
                        sem_in.at[c % NBUF],
                    )

                for c in range(min(NBUF - 1, nsub)):
                    start_in(c)
                for c in range(nsub):
                    in_h[c].wait()
                    out_h[c] = pltpu.async_copy(
                        buf.at[pl.ds((c % NBUF) * sub, sub)],
                        out_hbm.at[d, c],
                        sem_out.at[c % NBUF],
                    )
                    j = c + NBUF - 1
                    if j < nsub:
                        if j >= NBUF:
                            out_h[j - NBUF].wait()
                        start_in(j)
                for c in range(max(0, nsub - NBUF), nsub):
                    out_h[c].wait()

    return scatter(aux, frames3)


def kernel(frames, mask):
    T = frames.shape[0]
    row = 1
    for d in frames.shape[1:]:
        row *= d
    sub = row // SUBLANES
    frames3 = frames.reshape(T, SUBLANES, sub)
    src = _src_indices(T)

    idx = jnp.arange(T, dtype=jnp.int32)
    keep = src == idx
    n_keep = jnp.sum(keep.astype(jnp.int32))
    n_drop = jnp.int32(T) - n_keep
    big = jnp.int32(T)
    order = jnp.argsort(jnp.where(keep, idx, big + idx)).astype(jnp.int32)
    drop_order = jnp.roll(order, n_drop - big)
    first_keep = order[0]
    m = jnp.where(idx < n_keep, order, first_keep).astype(jnp.int32)

    aux_len = ((2 * T + 1 + LANES - 1) // LANES) * LANES
    aux = jnp.zeros((aux_len,), jnp.int32)
    aux = aux.at[:T].set(src).at[T:2 * T].set(drop_order)
    aux = aux.at[2 * T].set(n_drop)

    partial3 = _sc_scatter(aux, frames3)

    NB = 8
    D = 6

    def tc_copy(m_ref, in_ref, partial_ref, o_ref, buf, sem_in, sem_out):
        del partial_ref
        in_h = [None] * T
        out_h = [None] * T

        def start_in(k):
            in_h[k] = pltpu.make_async_copy(
                in_ref.at[m_ref[k]], buf.at[k % NB], sem_in.at[k % NB])
            in_h[k].start()

        for k in range(min(D, T)):
            start_in(k)
        for k in range(T):
            in_h[k].wait()
            out_h[k] = pltpu.make_async_copy(
                buf.at[k % NB], o_ref.at[m_ref[k]], sem_out.at[k % NB])
            out_h[k].start()
            j = k + D
            if j < T:
                if j >= NB:
                    out_h[j - NB].wait()
                start_in(j)
        for k in range(max(0, T - NB), T):
            out_h[k].wait()

    grid_spec = pltpu.PrefetchScalarGridSpec(
        num_scalar_prefetch=1,
        grid=(1,),
        in_specs=[
            pl.BlockSpec(memory_space=pl.ANY),
            pl.BlockSpec(memory_space=pl.ANY),
        ],
        out_specs=pl.BlockSpec(memory_space=pl.ANY),
        scratch_shapes=[
            pltpu.VMEM((NB, SUBLANES, sub), frames3.dtype),
            pltpu.SemaphoreType.DMA((NB,)),
            pltpu.SemaphoreType.DMA((NB,)),
        ],
    )
    out = pl.pallas_call(
        tc_copy,
        grid_spec=grid_spec,
        out_shape=jax.ShapeDtypeStruct((T, SUBLANES, sub), frames3.dtype),
        input_output_aliases={2: 0},
    )(m, frames3, partial3)

    return (out.reshape(frames.shape), mask)


# speedup vs baseline: 10.8626x; 1.0036x over previous
import dataclasses
import functools

import jax
import jax.numpy as jnp
from jax import lax
from jax.experimental import pallas as pl
from jax.experimental.pallas import tpu as pltpu
from jax.experimental.pallas import tpu_sc as plsc

DROP_FRAME_PROB = 0.125
NUM_SC_CORES = 2
NUM_SC_SUBCORES = 16
LANES = 16
NBUF = 4
SUBLANES = 8


def _src_indices(T):
    rkey = jax.random.key(42)
    kdrop, kdir = jax.random.split(rkey)
    u_drop = jax.random.uniform(kdrop, (T,))
    u_dir = jax.random.uniform(kdir, (T,))
    drop = u_drop < DROP_FRAME_PROB
    diff = jnp.where(u_dir < 0.5, -1, 1)
    idx = jnp.arange(T)
    return jnp.where(drop, (idx + diff) % T, idx).astype(jnp.int32)


def _sc_scatter(aux, frames3):
    T, nsub, sub = frames3.shape
    n_workers = NUM_SC_CORES * NUM_SC_SUBCORES
    aux_len = aux.shape[0]
    max_items = (T + n_workers - 1) // n_workers

    mesh = plsc.VectorSubcoreMesh(core_axis_name="c", subcore_axis_name="s")

    cp = pltpu.CompilerParams()
    if "needs_layout_passes" in pltpu.CompilerParams.__dataclass_fields__:
        cp = dataclasses.replace(cp, needs_layout_passes=False)

    @functools.partial(
        pl.kernel,
        compiler_params=cp,
        out_type=jax.ShapeDtypeStruct((T, nsub, sub), frames3.dtype),
        mesh=mesh,
        scratch_types=[
            pltpu.VMEM((aux_len,), jnp.int32),
            pltpu.VMEM((NBUF * sub,), frames3.dtype),
            pltpu.SemaphoreType.DMA,
            pltpu.SemaphoreType.DMA((NBUF,)),
            pltpu.SemaphoreType.DMA((NBUF,)),
        ],
    )
    def scatter(aux_hbm, frames_hbm, out_hbm, aux_v, buf, sem_i, sem_in,
                sem_out):
        wid = lax.axis_index("s") * NUM_SC_CORES + lax.axis_index("c")
        pltpu.async_copy(aux_hbm, aux_v, sem_i).wait()

        iota = lax.iota(jnp.int32, LANES)
        neg = jnp.int32(-(2**31) + 1)
        vecs = [aux_v[pl.ds(j * LANES, LANES)] for j in range(aux_len // LANES)]

        def extract(pos):
            s = neg
            for j, vec in enumerate(vecs):
                m = (iota + (j * LANES)) == pos
                s = jnp.maximum(s, jnp.max(jnp.where(m, vec, neg)))
            return s

        n_drop = extract(jnp.int32(2 * T))
        for t in range(max_items):
            k = wid + t * n_workers

            @pl.when(k < n_drop)
            def _():
                d = extract(T + k)
                s = extract(d)
                in_h = [None] * nsub
                out_h = [None] * nsub

                def start_in(c):
                    in_h[c] = pltpu.async_copy(
                        frames_hbm.at[s, c],
                        buf.at[pl.ds((c % NBUF) * sub, sub)],
                        sem_in.at[c % NBUF],
                    )

                for c in range(min(NBUF - 1, nsub)):
                    start_in(c)
                for c in range(nsub):
                    in_h[c].wait()
                    out_h[c] = pltpu.async_copy(
                        buf.at[pl.ds((c % NBUF) * sub, sub)],
                        out_hbm.at[d, c],
                        sem_out.at[c % NBUF],
                    )
                    j = c + NBUF - 1
                    if j < nsub:
                        if j >= NBUF:
                            out_h[j - NBUF].wait()
                        start_in(j)
                for c in range(max(0, nsub - NBUF), nsub):
                    out_h[c].wait()

    return scatter(aux, frames3)


def kernel(frames, mask):
    T = frames.shape[0]
    row = 1
    for d in frames.shape[1:]:
        row *= d
    sub = row // SUBLANES
    frames3 = frames.reshape(T, SUBLANES, sub)
    src = _src_indices(T)

    idx = jnp.arange(T, dtype=jnp.int32)
    keep = src == idx
    n_keep = jnp.sum(keep.astype(jnp.int32))
    n_drop = jnp.int32(T) - n_keep
    big = jnp.int32(T)
    order = jnp.argsort(jnp.where(keep, idx, big + idx)).astype(jnp.int32)
    drop_order = jnp.roll(order, n_drop - big)
    first_keep = order[0]
    m = jnp.where(idx < n_keep, order, first_keep).astype(jnp.int32)

    aux_len = ((2 * T + 1 + LANES - 1) // LANES) * LANES
    aux = jnp.zeros((aux_len,), jnp.int32)
    aux = aux.at[:T].set(src).at[T:2 * T].set(drop_order)
    aux = aux.at[2 * T].set(n_drop)

    partial3 = _sc_scatter(aux, frames3)

    NB = 16
    D = 6

    def tc_copy(m_ref, in_ref, partial_ref, o_ref, buf, sem_in, sem_out):
        del partial_ref
        in_h = [None] * T
        out_h = [None] * T

        def start_in(k):
            in_h[k] = pltpu.make_async_copy(
                in_ref.at[m_ref[k]], buf.at[k % NB], sem_in.at[k % NB])
            in_h[k].start()

        for k in range(min(D, T)):
            start_in(k)
        for k in range(T):
            in_h[k].wait()
            out_h[k] = pltpu.make_async_copy(
                buf.at[k % NB], o_ref.at[m_ref[k]], sem_out.at[k % NB])
            out_h[k].start()
            j = k + D
            if j < T:
                if j >= NB:
                    out_h[j - NB].wait()
                start_in(j)
        for k in range(max(0, T - NB), T):
            out_h[k].wait()

    grid_spec = pltpu.PrefetchScalarGridSpec(
        num_scalar_prefetch=1,
        grid=(1,),
        in_specs=[
            pl.BlockSpec(memory_space=pl.ANY),
            pl.BlockSpec(memory_space=pl.ANY),
        ],
        out_specs=pl.BlockSpec(memory_space=pl.ANY),
        scratch_shapes=[
            pltpu.VMEM((NB, SUBLANES, sub), frames3.dtype),
            pltpu.SemaphoreType.DMA((NB,)),
            pltpu.SemaphoreType.DMA((NB,)),
        ],
    )
    out = pl.pallas_call(
        tc_copy,
        grid_spec=grid_spec,
        out_shape=jax.ShapeDtypeStruct((T, SUBLANES, sub), frames3.dtype),
        input_output_aliases={2: 0},
    )(m, frames3, partial3)

    return (out.reshape(frames.shape), mask)
